# baseline (speedup 1.0000x reference)
import functools

import jax
import jax.numpy as jnp
from jax import lax
from jax.experimental import pallas as pl
from jax.experimental.pallas import tpu as pltpu
from jax.experimental.pallas import tpu_sc as plsc

_N = 26
_B = 4096
_K = 16
_NC = 2
_NS = 16
_NW = _NC * _NS
_CHUNK = _B // _NW


def _body(*refs):
    idx = refs[:_N]
    emb = refs[_N:2 * _N]
    msk = refs[2 * _N:3 * _N]
    out = refs[3 * _N]
    idx_v, erow, mrow, out_v, sem = refs[3 * _N + 1:]

    wid = lax.axis_index("s") * _NC + lax.axis_index("c")
    base = wid * _CHUNK

    for i in range(_N):
        pltpu.sync_copy(idx[i].at[pl.ds(base, _CHUNK)], idx_v)
        c1 = pltpu.async_copy(emb[i].at[idx_v], erow, sem)
        c2 = pltpu.async_copy(msk[i].at[idx_v], mrow, sem)
        c1.wait()
        c2.wait()

        def mul_body(r, _, i=i):
            out_v[r, i] = erow[r] * mrow[r]
            return 0

        lax.fori_loop(0, _CHUNK, mul_body, 0, unroll=4)

    pltpu.sync_copy(out_v, out.at[pl.ds(base, _CHUNK)])


_mesh = plsc.VectorSubcoreMesh(
    core_axis_name="c", subcore_axis_name="s", num_cores=_NC, num_subcores=_NS
)

_lookup = pl.kernel(
    _body,
    out_type=jax.ShapeDtypeStruct((_B, _N, _K), jnp.float32),
    mesh=_mesh,
    scratch_types=[
        pltpu.VMEM((_CHUNK,), jnp.int32),
        pltpu.VMEM((_CHUNK, _K), jnp.float32),
        pltpu.VMEM((_CHUNK, _K), jnp.float32),
        pltpu.VMEM((_CHUNK, _N, _K), jnp.float32),
        pltpu.SemaphoreType.DMA,
    ],
    compiler_params=pltpu.CompilerParams(use_tc_tiling_on_sc=False),
)


def kernel(*args):
    idxs = [jnp.reshape(a, (_B,)).astype(jnp.int32) for a in args[:_N]]
    embs = args[_N:2 * _N]
    msks = args[2 * _N:3 * _N]
    return _lookup(*idxs, *embs, *msks)


# speedup vs baseline: 2.4323x; 2.4323x over previous
import jax
import jax.numpy as jnp
from jax import lax
from jax.experimental import pallas as pl
from jax.experimental.pallas import tpu as pltpu
from jax.experimental.pallas import tpu_sc as plsc

_N = 26
_B = 4096
_K = 16
_V = 100000
_L = 16
_CH = 1792
_NR = 7
_VA = 99968
_T = _V - _VA


def _round(tab_e, tab_m, shr_e, shr_m, col_v, c, s, pair, is_val, base):
    q = s % 8

    @pl.when(s < 8)
    def _():
        pltpu.sync_copy(
            tab_e.at[pl.ds(c * 8, 8), pl.ds(base + q * _CH, _CH)],
            shr_e.at[q],
        )

    @pl.when(s >= 8)
    def _():
        pltpu.sync_copy(
            tab_m.at[pl.ds(c * 8, 8), pl.ds(base + q * _CH, _CH)],
            shr_m.at[q],
        )

    plsc.subcore_barrier()

    def pull_body(qq, _):
        @pl.when(is_val)
        def _():
            pltpu.sync_copy(shr_e.at[qq, pair],
                            col_v.at[pl.ds(base + qq * _CH, _CH)])

        @pl.when(jnp.logical_not(is_val))
        def _():
            pltpu.sync_copy(shr_m.at[qq, pair],
                            col_v.at[pl.ds(base + qq * _CH, _CH)])

        return 0

    lax.fori_loop(0, 8, pull_body, 0)
    plsc.subcore_barrier()


def _body(*refs):
    idx = refs[:_N]
    emb = refs[_N:2 * _N]
    msk = refs[2 * _N:3 * _N]
    tle = refs[3 * _N]
    tlm = refs[3 * _N + 1]
    out = refs[3 * _N + 2]
    (idx_v, col_v, val_v, prt_v, tail_v,
     shr_e, shr_m, mail, prod, sem) = refs[3 * _N + 3:]

    c = lax.axis_index("c")
    s = lax.axis_index("s")
    pair = s // 2
    is_val = s % 2 == 0
    is_msk = s % 2 == 1

    for i in range(_N):
        pltpu.sync_copy(idx[i], idx_v)
        tbase = (i * _K + c * 8 + pair) * _T

        @pl.when(is_val)
        def _():
            pltpu.sync_copy(tle.at[pl.ds(tbase, _T)], tail_v)

        @pl.when(is_msk)
        def _():
            pltpu.sync_copy(tlm.at[pl.ds(tbase, _T)], tail_v)

        def round_body(r, _, ei=emb[i], mi=msk[i]):
            base = jnp.minimum(r * 8 * _CH, _VA - 8 * _CH)
            _round(ei, mi, shr_e, shr_m, col_v, c, s, pair, is_val, base)
            return 0

        lax.fori_loop(0, _NR, round_body, 0)

        def gather_body(r, _):
            sl = pl.ds(r * _L, _L)
            iv = idx_v[sl]
            g = plsc.load_gather(col_v, [jnp.minimum(iv, _VA - 1)])
            gt = plsc.load_gather(tail_v, [jnp.maximum(iv - _VA, 0)])
            val_v[sl] = jnp.where(iv >= _VA, gt, g)
            return 0

        lax.fori_loop(0, _B // _L, gather_body, 0, unroll=4)

        @pl.when(is_msk)
        def _():
            pltpu.sync_copy(val_v, mail.at[0, pair])

        plsc.subcore_barrier()

        @pl.when(is_val)
        def _():
            pltpu.sync_copy(mail.at[0, pair], prt_v)

            def mul_body(tb, _):
                for u in range(8):
                    sl = pl.ds(tb * 128 + u * _L, _L)
                    val_v[sl] = val_v[sl] * prt_v[sl]
                pltpu.sync_copy(val_v.at[pl.ds(tb * 128, 128)],
                                prod.at[tb, pair])
                return 0

            lax.fori_loop(0, _B // 128, mul_body, 0)

        plsc.subcore_barrier()

        @pl.when(s == 0)
        def _():
            pltpu.sync_copy(prod, out.at[i, c])


_mesh = plsc.VectorSubcoreMesh(
    core_axis_name="c", subcore_axis_name="s", num_cores=2, num_subcores=16
)

_lookup = pl.kernel(
    _body,
    out_type=jax.ShapeDtypeStruct((_N, 2, _B // 128, 8, 128), jnp.float32),
    mesh=_mesh,
    scratch_types=[
        pltpu.VMEM((_B,), jnp.int32),
        pltpu.VMEM((_VA,), jnp.float32),
        pltpu.VMEM((_B,), jnp.float32),
        pltpu.VMEM((_B,), jnp.float32),
        pltpu.VMEM((_T,), jnp.float32),
        pltpu.VMEM_SHARED((8, 8, _CH), jnp.float32),
        pltpu.VMEM_SHARED((8, 8, _CH), jnp.float32),
        pltpu.VMEM_SHARED((1, 8, _B), jnp.float32),
        pltpu.VMEM_SHARED((_B // 128, 8, 128), jnp.float32),
        pltpu.SemaphoreType.DMA,
    ],
    compiler_params=pltpu.CompilerParams(needs_layout_passes=False),
)


def kernel(*args):
    idxs = [jnp.reshape(a, (_B,)).astype(jnp.int32) for a in args[:_N]]
    embs = [jnp.transpose(t) for t in args[_N:2 * _N]]
    msks = [jnp.transpose(t) for t in args[2 * _N:3 * _N]]
    tle = jnp.reshape(jnp.stack([t[:, _VA:_V] for t in embs]), (-1,))
    tlm = jnp.reshape(jnp.stack([t[:, _VA:_V] for t in msks]), (-1,))
    out5 = _lookup(*idxs, *embs, *msks, tle, tlm)
    return jnp.reshape(jnp.transpose(out5, (2, 4, 0, 1, 3)), (_B, _N, _K))


# speedup vs baseline: 3.0468x; 1.2527x over previous
import jax
import jax.numpy as jnp
from jax import lax
from jax.experimental import pallas as pl
from jax.experimental.pallas import tpu as pltpu
from jax.experimental.pallas import tpu_sc as plsc

_N = 26
_B = 4096
_K = 16
_V = 100000
_L = 16
_CH = 1792
_NR = 7
_VA = 99968
_T = _V - _VA


def _round(tab_e, tab_m, shr_e, shr_m, col_v, c, s, pair, is_val, base,
           idx0, idx_v, sem):
    q = s % 8

    @pl.when(s < 8)
    def _():
        pltpu.sync_copy(
            tab_e.at[pl.ds(c * 8, 8), pl.ds(base + q * _CH, _CH)],
            shr_e.at[q],
        )

    @pl.when(s >= 8)
    def _():
        pltpu.sync_copy(
            tab_m.at[pl.ds(c * 8, 8), pl.ds(base + q * _CH, _CH)],
            shr_m.at[q],
        )

    plsc.subcore_barrier()

    def pull_body(qq, _):
        @pl.when(is_val)
        def _():
            pltpu.async_copy(shr_e.at[qq, pair],
                             col_v.at[pl.ds(base + qq * _CH, _CH)], sem)

        @pl.when(jnp.logical_not(is_val))
        def _():
            pltpu.async_copy(shr_m.at[qq, pair],
                             col_v.at[pl.ds(base + qq * _CH, _CH)], sem)

        return 0

    lax.fori_loop(0, 8, pull_body, 0)
    for _i in range(3):
        pltpu.make_async_copy(idx0, idx_v, sem).wait()
    pltpu.make_async_copy(idx0.at[pl.ds(0, 2048)],
                          idx_v.at[pl.ds(0, 2048)], sem).wait()
    plsc.subcore_barrier()


def _body(*refs):
    idx = refs[:_N]
    emb = refs[_N:2 * _N]
    msk = refs[2 * _N:3 * _N]
    tle = refs[3 * _N]
    tlm = refs[3 * _N + 1]
    out = refs[3 * _N + 2]
    (idx_v, col_v, val_v, prt_v, tail_v,
     shr_e, shr_m, mail, prod, sem) = refs[3 * _N + 3:]

    c = lax.axis_index("c")
    s = lax.axis_index("s")
    pair = s // 2
    is_val = s % 2 == 0
    is_msk = s % 2 == 1

    for i in range(_N):
        pltpu.sync_copy(idx[i], idx_v)
        tbase = (i * _K + c * 8 + pair) * _T

        @pl.when(is_val)
        def _():
            pltpu.sync_copy(tle.at[pl.ds(tbase, _T)], tail_v)

        @pl.when(is_msk)
        def _():
            pltpu.sync_copy(tlm.at[pl.ds(tbase, _T)], tail_v)

        def round_body(r, _, ei=emb[i], mi=msk[i]):
            base = jnp.minimum(r * 8 * _CH, _VA - 8 * _CH)
            _round(ei, mi, shr_e, shr_m, col_v, c, s, pair, is_val, base,
                   idx[0], idx_v, sem)
            return 0

        lax.fori_loop(0, _NR, round_body, 0)

        def gather_body(r, _):
            sl = pl.ds(r * _L, _L)
            iv = idx_v[sl]
            g = plsc.load_gather(col_v, [jnp.minimum(iv, _VA - 1)])
            gt = plsc.load_gather(tail_v, [jnp.maximum(iv - _VA, 0)])
            val_v[sl] = jnp.where(iv >= _VA, gt, g)
            return 0

        lax.fori_loop(0, _B // _L, gather_body, 0, unroll=4)

        @pl.when(is_msk)
        def _():
            pltpu.sync_copy(val_v, mail.at[0, pair])

        plsc.subcore_barrier()

        @pl.when(is_val)
        def _():
            pltpu.sync_copy(mail.at[0, pair], prt_v)

            def mul_body(tb, _):
                for u in range(8):
                    sl = pl.ds(tb * 128 + u * _L, _L)
                    val_v[sl] = val_v[sl] * prt_v[sl]
                pltpu.async_copy(val_v.at[pl.ds(tb * 128, 128)],
                                 prod.at[tb, pair], sem)
                return 0

            lax.fori_loop(0, _B // 128, mul_body, 0)
            pltpu.make_async_copy(idx[0], idx_v, sem).wait()

        plsc.subcore_barrier()

        @pl.when(s == 0)
        def _():
            pltpu.sync_copy(prod, out.at[i, c])


_mesh = plsc.VectorSubcoreMesh(
    core_axis_name="c", subcore_axis_name="s", num_cores=2, num_subcores=16
)

_lookup = pl.kernel(
    _body,
    out_type=jax.ShapeDtypeStruct((_N, 2, _B // 128, 8, 128), jnp.float32),
    mesh=_mesh,
    scratch_types=[
        pltpu.VMEM((_B,), jnp.int32),
        pltpu.VMEM((_VA,), jnp.float32),
        pltpu.VMEM((_B,), jnp.float32),
        pltpu.VMEM((_B,), jnp.float32),
        pltpu.VMEM((_T,), jnp.float32),
        pltpu.VMEM_SHARED((8, 8, _CH), jnp.float32),
        pltpu.VMEM_SHARED((8, 8, _CH), jnp.float32),
        pltpu.VMEM_SHARED((1, 8, _B), jnp.float32),
        pltpu.VMEM_SHARED((_B // 128, 8, 128), jnp.float32),
        pltpu.SemaphoreType.DMA,
    ],
    compiler_params=pltpu.CompilerParams(needs_layout_passes=False),
)


def kernel(*args):
    idxs = [jnp.reshape(a, (_B,)).astype(jnp.int32) for a in args[:_N]]
    embs = [jnp.transpose(t) for t in args[_N:2 * _N]]
    msks = [jnp.transpose(t) for t in args[2 * _N:3 * _N]]
    tle = jnp.reshape(jnp.stack([t[:, _VA:_V] for t in embs]), (-1,))
    tlm = jnp.reshape(jnp.stack([t[:, _VA:_V] for t in msks]), (-1,))
    out5 = _lookup(*idxs, *embs, *msks, tle, tlm)
    return jnp.reshape(jnp.transpose(out5, (2, 4, 0, 1, 3)), (_B, _N, _K))


# speedup vs baseline: 3.1530x; 1.0349x over previous
import jax
import jax.numpy as jnp
from jax import lax
from jax.experimental import pallas as pl
from jax.experimental.pallas import tpu as pltpu
from jax.experimental.pallas import tpu_sc as plsc

_N = 26
_B = 4096
_K = 16
_V = 100000
_L = 16
_CH = 1792
_NR = 7
_VA = 99968
_T = _V - _VA


def _round(tab_e, tab_m, shr_e, shr_m, col_v, c, s, pair, is_val, base,
           idx0, idx_v, sem):
    q = s % 8

    @pl.when(s < 8)
    def _():
        pltpu.sync_copy(
            tab_e.at[pl.ds(c * 8, 8), pl.ds(base + q * _CH, _CH)],
            shr_e.at[q],
        )

    @pl.when(s >= 8)
    def _():
        pltpu.sync_copy(
            tab_m.at[pl.ds(c * 8, 8), pl.ds(base + q * _CH, _CH)],
            shr_m.at[q],
        )

    plsc.subcore_barrier()

    def pull_body(qq, _):
        @pl.when(is_val)
        def _():
            pltpu.async_copy(shr_e.at[qq, pair],
                             col_v.at[pl.ds(base + qq * _CH, _CH)], sem)

        @pl.when(jnp.logical_not(is_val))
        def _():
            pltpu.async_copy(shr_m.at[qq, pair],
                             col_v.at[pl.ds(base + qq * _CH, _CH)], sem)

        return 0

    lax.fori_loop(0, 8, pull_body, 0)
    for _i in range(3):
        pltpu.make_async_copy(idx0, idx_v, sem).wait()
    pltpu.make_async_copy(idx0.at[pl.ds(0, 2048)],
                          idx_v.at[pl.ds(0, 2048)], sem).wait()
    plsc.subcore_barrier()


def _body(*refs):
    idx = refs[:_N]
    emb = refs[_N:2 * _N]
    msk = refs[2 * _N:3 * _N]
    tle = refs[3 * _N]
    tlm = refs[3 * _N + 1]
    out = refs[3 * _N + 2]
    (idx_v, col_v, val_v, prt_v, tail_v,
     shr_e, shr_m, mail, sem) = refs[3 * _N + 3:]

    c = lax.axis_index("c")
    s = lax.axis_index("s")
    pair = s // 2
    is_val = s % 2 == 0
    is_msk = s % 2 == 1

    for i in range(_N):
        pltpu.sync_copy(idx[i], idx_v)
        tbase = (i * _K + c * 8 + pair) * _T

        @pl.when(is_val)
        def _():
            pltpu.sync_copy(tle.at[pl.ds(tbase, _T)], tail_v)

        @pl.when(is_msk)
        def _():
            pltpu.sync_copy(tlm.at[pl.ds(tbase, _T)], tail_v)

        def round_body(r, _, ei=emb[i], mi=msk[i]):
            base = jnp.minimum(r * 8 * _CH, _VA - 8 * _CH)
            _round(ei, mi, shr_e, shr_m, col_v, c, s, pair, is_val, base,
                   idx[0], idx_v, sem)
            return 0

        lax.fori_loop(0, _NR, round_body, 0)

        def gather_body(r, _):
            sl = pl.ds(r * _L, _L)
            iv = idx_v[sl]
            g = plsc.load_gather(col_v, [jnp.minimum(iv, _VA - 1)])
            gt = plsc.load_gather(tail_v, [jnp.maximum(iv - _VA, 0)])
            val_v[sl] = jnp.where(iv >= _VA, gt, g)
            return 0

        lax.fori_loop(0, _B // _L, gather_body, 0, unroll=4)

        @pl.when(is_msk)
        def _():
            pltpu.sync_copy(val_v, mail.at[0, pair])

        plsc.subcore_barrier()

        @pl.when(is_val)
        def _():
            pltpu.sync_copy(mail.at[0, pair], prt_v)

            def mul_body(tb, _, i=i):
                for u in range(8):
                    sl = pl.ds(tb * 128 + u * _L, _L)
                    val_v[sl] = val_v[sl] * prt_v[sl]
                pltpu.async_copy(val_v.at[pl.ds(tb * 128, 128)],
                                 out.at[i, c, tb, pair], sem)
                return 0

            lax.fori_loop(0, _B // 128, mul_body, 0)
            pltpu.make_async_copy(idx[0], idx_v, sem).wait()


_mesh = plsc.VectorSubcoreMesh(
    core_axis_name="c", subcore_axis_name="s", num_cores=2, num_subcores=16
)

_lookup = pl.kernel(
    _body,
    out_type=jax.ShapeDtypeStruct((_N, 2, _B // 128, 8, 128), jnp.float32),
    mesh=_mesh,
    scratch_types=[
        pltpu.VMEM((_B,), jnp.int32),
        pltpu.VMEM((_VA,), jnp.float32),
        pltpu.VMEM((_B,), jnp.float32),
        pltpu.VMEM((_B,), jnp.float32),
        pltpu.VMEM((_T,), jnp.float32),
        pltpu.VMEM_SHARED((8, 8, _CH), jnp.float32),
        pltpu.VMEM_SHARED((8, 8, _CH), jnp.float32),
        pltpu.VMEM_SHARED((1, 8, _B), jnp.float32),
        pltpu.SemaphoreType.DMA,
    ],
    compiler_params=pltpu.CompilerParams(needs_layout_passes=False),
)


def kernel(*args):
    idxs = [jnp.reshape(a, (_B,)).astype(jnp.int32) for a in args[:_N]]
    embs = [jnp.transpose(t) for t in args[_N:2 * _N]]
    msks = [jnp.transpose(t) for t in args[2 * _N:3 * _N]]
    tle = jnp.reshape(jnp.stack([t[:, _VA:_V] for t in embs]), (-1,))
    tlm = jnp.reshape(jnp.stack([t[:, _VA:_V] for t in msks]), (-1,))
    out5 = _lookup(*idxs, *embs, *msks, tle, tlm)
    return jnp.reshape(jnp.transpose(out5, (2, 4, 0, 1, 3)), (_B, _N, _K))


# speedup vs baseline: 3.4066x; 1.0804x over previous
import jax
import jax.numpy as jnp
from jax import lax
from jax.experimental import pallas as pl
from jax.experimental.pallas import tpu as pltpu
from jax.experimental.pallas import tpu_sc as plsc

_N = 26
_B = 4096
_K = 16
_V = 100000
_L = 16
_CH = 1792
_NR = 7
_VA = 99968
_T = _V - _VA


def _round(tab_e, tab_m, shr_e, shr_m, col_v, c, s, pair, is_val, base,
           idx0, idx_v, sem):
    q = s % 8

    @pl.when(s < 8)
    def _():
        pltpu.sync_copy(
            tab_e.at[pl.ds(c * 8, 8), pl.ds(base + q * _CH, _CH)],
            shr_e.at[q],
        )

    @pl.when(s >= 8)
    def _():
        pltpu.sync_copy(
            tab_m.at[pl.ds(c * 8, 8), pl.ds(base + q * _CH, _CH)],
            shr_m.at[q],
        )

    plsc.subcore_barrier()

    def pull_body(qq, _):
        @pl.when(is_val)
        def _():
            pltpu.async_copy(shr_e.at[qq, pair],
                             col_v.at[pl.ds(base + qq * _CH, _CH)], sem)

        @pl.when(jnp.logical_not(is_val))
        def _():
            pltpu.async_copy(shr_m.at[qq, pair],
                             col_v.at[pl.ds(base + qq * _CH, _CH)], sem)

        return 0

    lax.fori_loop(0, 8, pull_body, 0)
    for _i in range(3):
        pltpu.make_async_copy(idx0, idx_v, sem).wait()
    pltpu.make_async_copy(idx0.at[pl.ds(0, 2048)],
                          idx_v.at[pl.ds(0, 2048)], sem).wait()
    plsc.subcore_barrier()


def _body(*refs):
    idx = refs[:_N]
    emb = refs[_N:2 * _N]
    msk = refs[2 * _N:3 * _N]
    tle = refs[3 * _N]
    tlm = refs[3 * _N + 1]
    out = refs[3 * _N + 2]
    (idx_v, col_v, val_v, prt_v,
     shr_e, shr_m, mail, sem, sem2) = refs[3 * _N + 3:]

    c = lax.axis_index("c")
    s = lax.axis_index("s")
    pair = s // 2
    is_val = s % 2 == 0
    is_msk = s % 2 == 1

    for i in range(_N):
        pltpu.async_copy(idx[i], idx_v, sem2)
        tbase = (i * _K + c * 8 + pair) * _T

        @pl.when(is_val)
        def _():
            pltpu.async_copy(tle.at[pl.ds(tbase, _T)],
                             col_v.at[pl.ds(_VA, _T)], sem2)

        @pl.when(is_msk)
        def _():
            pltpu.async_copy(tlm.at[pl.ds(tbase, _T)],
                             col_v.at[pl.ds(_VA, _T)], sem2)

        def round_body(r, _, ei=emb[i], mi=msk[i]):
            base = jnp.minimum(r * 8 * _CH, _VA - 8 * _CH)
            _round(ei, mi, shr_e, shr_m, col_v, c, s, pair, is_val, base,
                   idx[0], idx_v, sem)
            return 0

        lax.fori_loop(0, _NR, round_body, 0)

        pltpu.make_async_copy(idx[0], idx_v, sem2).wait()
        pltpu.make_async_copy(tle.at[pl.ds(0, _T)],
                              col_v.at[pl.ds(_VA, _T)], sem2).wait()

        def gather_body(r, _):
            sl = pl.ds(r * _L, _L)
            val_v[sl] = plsc.load_gather(col_v, [idx_v[sl]])
            return 0

        lax.fori_loop(0, _B // _L, gather_body, 0, unroll=8)

        @pl.when(is_msk)
        def _():
            pltpu.sync_copy(val_v, mail.at[0, pair])

        plsc.subcore_barrier()

        @pl.when(is_val)
        def _():
            pltpu.sync_copy(mail.at[0, pair], prt_v)

            def mul_body(tb, _, i=i):
                for u in range(8):
                    sl = pl.ds(tb * 128 + u * _L, _L)
                    val_v[sl] = val_v[sl] * prt_v[sl]
                pltpu.async_copy(val_v.at[pl.ds(tb * 128, 128)],
                                 out.at[i, c, tb, pair], sem)
                return 0

            lax.fori_loop(0, _B // 128, mul_body, 0)
            pltpu.make_async_copy(idx[0], idx_v, sem).wait()


_mesh = plsc.VectorSubcoreMesh(
    core_axis_name="c", subcore_axis_name="s", num_cores=2, num_subcores=16
)

_lookup = pl.kernel(
    _body,
    out_type=jax.ShapeDtypeStruct((_N, 2, _B // 128, 8, 128), jnp.float32),
    mesh=_mesh,
    scratch_types=[
        pltpu.VMEM((_B,), jnp.int32),
        pltpu.VMEM((_V,), jnp.float32),
        pltpu.VMEM((_B,), jnp.float32),
        pltpu.VMEM((_B,), jnp.float32),
        pltpu.VMEM_SHARED((8, 8, _CH), jnp.float32),
        pltpu.VMEM_SHARED((8, 8, _CH), jnp.float32),
        pltpu.VMEM_SHARED((1, 8, _B), jnp.float32),
        pltpu.SemaphoreType.DMA,
        pltpu.SemaphoreType.DMA,
    ],
    compiler_params=pltpu.CompilerParams(needs_layout_passes=False),
)


def kernel(*args):
    idxs = [jnp.reshape(a, (_B,)).astype(jnp.int32) for a in args[:_N]]
    embs = [jnp.transpose(t) for t in args[_N:2 * _N]]
    msks = [jnp.transpose(t) for t in args[2 * _N:3 * _N]]
    tle = jnp.reshape(jnp.stack([t[:, _VA:_V] for t in embs]), (-1,))
    tlm = jnp.reshape(jnp.stack([t[:, _VA:_V] for t in msks]), (-1,))
    out5 = _lookup(*idxs, *embs, *msks, tle, tlm)
    return jnp.reshape(jnp.transpose(out5, (2, 4, 0, 1, 3)), (_B, _N, _K))
